# baseline (speedup 1.0000x reference)
import functools

import jax
import jax.numpy as jnp
from jax import lax
from jax.experimental import pallas as pl
from jax.experimental.pallas import tpu as pltpu
from jax.experimental.pallas import tpu_sc as plsc

N_NODES = 10000
N_EDGES = 160000
IN_CH = 256
HID_CH = 256
OUT_CH = 128

NC = 2
NS = 16
L = 16
NW = NC * NS

HP = 10240
DUMP = N_NODES
EC = 128
EP = 163840

RB = 1024


def _sc_mesh():
    return plsc.VectorSubcoreMesh(core_axis_name="c", subcore_axis_name="s")


_CB = 128


def _eprep_body(ei_ref, ep_ref):
    b = pl.program_id(0)
    e = ei_ref[...].reshape(2, _CB, EC)
    spare = HP - N_NODES
    ch = jax.lax.broadcasted_iota(jnp.int32, (2, _CB, EC), 1)
    lane = jax.lax.broadcasted_iota(jnp.int32, (2, _CB, EC), 2)
    eid = (b * _CB + ch) * EC + lane
    p = eid - N_EDGES
    row = jax.lax.broadcasted_iota(jnp.int32, (2, _CB, EC), 0)
    pad_val = DUMP + ((p + jnp.where(row == 0, spare // 2, 0)) % spare)
    ep_ref[...] = jnp.where(eid >= N_EDGES, pad_val, e)


def _eprep(edge_index):
    nch = EP // EC
    grid = (nch // _CB,)
    return pl.pallas_call(
        _eprep_body,
        grid=grid,
        in_specs=[pl.BlockSpec((2, _CB * EC), lambda i: (0, i))],
        out_specs=pl.BlockSpec((2, _CB, EC), lambda i: (0, i, 0)),
        out_shape=jax.ShapeDtypeStruct((2, nch, EC), jnp.int32),
    )(edge_index)


def _make_deg():
    epw = EP // NW
    nchunks = epw // EC

    @functools.partial(
        pl.kernel,
        out_type=jax.ShapeDtypeStruct((NW, HP), jnp.float32),
        mesh=_sc_mesh(),
        scratch_types=[
            pltpu.VMEM((2 * G, EC), jnp.int32),
            pltpu.VMEM((HP,), jnp.float32),
            pltpu.SemaphoreType.DMA,
        ],
        compiler_params=pltpu.CompilerParams(needs_layout_passes=False),
    )
    def deg_kernel(epairs, deg_hbm, dstb, hist, isem):
        c = lax.axis_index("c")
        s = lax.axis_index("s")
        wid = c * NS + s
        ch0 = wid * nchunks
        ng = nchunks // G
        zeros = jnp.zeros((L,), jnp.float32)

        def zero_body(i, carry):
            for j in range(8):
                hist[pl.ds((i * 8 + j) * L, L)] = zeros
            return carry

        lax.fori_loop(0, HP // L // 8, zero_body, 0)

        ones = jnp.ones((L,), jnp.float32)

        def idx_start(g, gb):
            pltpu.async_copy(epairs.at[1, pl.ds(ch0 + g * G, G)],
                             dstb.at[pl.ds(gb * G, G)], isem)

        def idx_wait(gb):
            pltpu.make_async_copy(epairs.at[1, pl.ds(ch0, G)],
                                  dstb.at[pl.ds(gb * G, G)], isem).wait()

        pltpu.sync_copy(epairs.at[1, pl.ds(ch0, G)],
                        dstb.at[pl.ds(0, G)])
        idx_start(1, 1)

        def group(m, gb, tail=False):
            for j in range(G):
                for l in range(EC // L):
                    idx = dstb[gb * G + j, pl.ds(l * L, L)]
                    plsc.addupdate_scatter(hist, [idx], ones)

            if not tail:
                @pl.when(m < ng - 1)
                def _():
                    idx_wait(gb ^ 1)

                @pl.when(m < ng - 2)
                def _():
                    idx_start(m + 2, gb)

        def body(mm, carry):
            group(2 * mm, 0)
            group(2 * mm + 1, 1)
            return carry

        lax.fori_loop(0, ng // 2, body, 0)
        if ng % 2:
            group(ng - 1, (ng - 1) % 2, tail=True)
        pltpu.sync_copy(hist, deg_hbm.at[wid])

    return deg_kernel


G = 8


def _edge_loop(h, epairs, acc, sidxb, didxb, rows, gsem0, gsem1,
               ssem0, ssem1, isem, chunk0, nchunks):
    gsems = (gsem0, gsem1)
    ssems = (ssem0, ssem1)
    ng = nchunks // G

    def idx_start(g, gb):
        pltpu.async_copy(epairs.at[0, pl.ds(chunk0 + g * G, G)],
                         sidxb.at[gb], isem)
        pltpu.async_copy(epairs.at[1, pl.ds(chunk0 + g * G, G)],
                         didxb.at[gb], isem)

    def idx_wait(gb):
        pltpu.make_async_copy(epairs.at[0, pl.ds(chunk0, G)],
                              sidxb.at[gb], isem).wait()
        pltpu.make_async_copy(epairs.at[1, pl.ds(chunk0, G)],
                              didxb.at[gb], isem).wait()

    def start_gather(b, gb, j):
        pltpu.async_copy(h.at[sidxb.at[gb, j]], rows.at[b], gsems[b])

    def wait_gather(b):
        pltpu.make_async_copy(h.at[sidxb.at[0, 0]], rows.at[b],
                              gsems[b]).wait()

    def start_scatter(b, gb, j):
        pltpu.async_copy(rows.at[b], acc.at[didxb.at[gb, j]], ssems[b],
                         add=True)

    def wait_scatter(b):
        pltpu.make_async_copy(rows.at[b], acc.at[didxb.at[0, 0]],
                              ssems[b]).wait()

    def maybe_when(cond, fn):
        if isinstance(cond, bool):
            if cond:
                fn()
        else:
            pl.when(cond)(fn)

    pltpu.sync_copy(epairs.at[0, pl.ds(chunk0, G)], sidxb.at[0])
    pltpu.sync_copy(epairs.at[1, pl.ds(chunk0, G)], didxb.at[0])
    idx_start(1, 1)
    start_gather(0, 0, 0)

    def group(m, gb, first=False, tail=False):
        for j in range(G):
            rb = j & 1
            wait_gather(rb)
            if j < G - 1:
                if not (first and j == 0):
                    wait_scatter(rb ^ 1)
                start_gather(rb ^ 1, gb, j + 1)
            elif not tail:
                def _boundary():
                    idx_wait(gb ^ 1)
                    wait_scatter(rb ^ 1)
                    start_gather(rb ^ 1, gb ^ 1, 0)
                maybe_when(m < ng - 1, _boundary)
            start_scatter(rb, gb, j)

        if not tail:
            maybe_when(m < ng - 2, lambda: idx_start(m + 2, gb))

    group(0, 0, first=True)

    def body(mm, carry):
        group(1 + 2 * mm, 1)
        group(2 + 2 * mm, 0)
        return carry

    lax.fori_loop(0, (ng - 1) // 2, body, 0)
    if ng % 2 == 0:
        group(ng - 1, (ng - 1) % 2, tail=True)
    wait_scatter(0)
    wait_scatter(1)


def _agg_scratch(F):
    return [
        pltpu.VMEM((2, G, EC), jnp.int32),
        pltpu.VMEM((2, G, EC), jnp.int32),
        pltpu.VMEM((2, EC, F), jnp.float32),
        pltpu.VMEM_SHARED((HP, F), jnp.float32),
        pltpu.SemaphoreType.DMA,
        pltpu.SemaphoreType.DMA,
        pltpu.SemaphoreType.DMA,
        pltpu.SemaphoreType.DMA,
        pltpu.SemaphoreType.DMA,
    ]


def _make_agg(F):
    rpw = HP // NS
    nchunks = EP // EC // NS

    @functools.partial(
        pl.kernel,
        out_type=(jax.ShapeDtypeStruct((HP, F), jnp.float32),
                  jax.ShapeDtypeStruct((HP, F), jnp.float32)),
        mesh=_sc_mesh(),
        scratch_types=_agg_scratch(F),
    )
    def agg_kernel(h0, h1, zrows, epairs, o0, o1, sidxb, didxb, rows, acc,
                   gsem0, gsem1, ssem0, ssem1, isem):
        c = lax.axis_index("c")
        s = lax.axis_index("s")
        r0 = s * rpw
        chunk0 = s * nchunks

        pltpu.sync_copy(zrows, acc.at[pl.ds(r0, rpw)])
        plsc.subcore_barrier()

        @pl.when(c == 0)
        def _():
            _edge_loop(h0, epairs, acc, sidxb, didxb, rows, gsem0, gsem1,
                       ssem0, ssem1, isem, chunk0, nchunks)

        @pl.when(c == 1)
        def _():
            _edge_loop(h1, epairs, acc, sidxb, didxb, rows, gsem0, gsem1,
                       ssem0, ssem1, isem, chunk0, nchunks)

        plsc.subcore_barrier()

        def writeback(o):
            pltpu.sync_copy(acc.at[pl.ds(r0, rpw)], o.at[pl.ds(r0, rpw)])

        @pl.when(c == 0)
        def _():
            writeback(o0)

        @pl.when(c == 1)
        def _():
            writeback(o1)

    return agg_kernel


def _make_agg_edge_split(F):
    rpw = HP // NS
    nchunks = EP // EC // NW

    @functools.partial(
        pl.kernel,
        out_type=(jax.ShapeDtypeStruct((HP, F), jnp.float32),
                  jax.ShapeDtypeStruct((HP, F), jnp.float32)),
        mesh=_sc_mesh(),
        scratch_types=_agg_scratch(F),
    )
    def agg_kernel(g, zrows, epairs, o0, o1, sidxb, didxb, rows, acc,
                   gsem0, gsem1, ssem0, ssem1, isem):
        c = lax.axis_index("c")
        s = lax.axis_index("s")
        wid = c * NS + s
        r0 = s * rpw

        pltpu.sync_copy(zrows, acc.at[pl.ds(r0, rpw)])
        plsc.subcore_barrier()
        _edge_loop(g, epairs, acc, sidxb, didxb, rows, gsem0, gsem1,
                   ssem0, ssem1, isem, wid * nchunks, nchunks)
        plsc.subcore_barrier()

        @pl.when(c == 0)
        def _():
            pltpu.sync_copy(acc.at[pl.ds(r0, rpw)], o0.at[pl.ds(r0, rpw)])

        @pl.when(c == 1)
        def _():
            pltpu.sync_copy(acc.at[pl.ds(r0, rpw)], o1.at[pl.ds(r0, rpw)])

    return agg_kernel


def _dinv_of(deg_blk):
    return lax.rsqrt(1.0 + jnp.sum(deg_blk, axis=0))


def _mm1_body(x_ref, w_ref, deg_ref, h0_ref, h1_ref):
    dinv = _dinv_of(deg_ref[...])
    h = jnp.dot(x_ref[...], w_ref[...], preferred_element_type=jnp.float32)
    h = h * dinv[:, None]
    h0_ref[...] = h[:, :HID_CH // 2]
    h1_ref[...] = h[:, HID_CH // 2:]


def _mm1(x_p, W1, deg_parts):
    grid = (HP // RB,)
    return pl.pallas_call(
        _mm1_body,
        grid=grid,
        in_specs=[
            pl.BlockSpec((RB, IN_CH), lambda i: (i, 0)),
            pl.BlockSpec((IN_CH, HID_CH), lambda i: (0, 0)),
            pl.BlockSpec((NW, RB), lambda i: (0, i)),
        ],
        out_specs=[
            pl.BlockSpec((RB, HID_CH // 2), lambda i: (i, 0)),
            pl.BlockSpec((RB, HID_CH // 2), lambda i: (i, 0)),
        ],
        out_shape=[
            jax.ShapeDtypeStruct((HP, HID_CH // 2), jnp.float32),
            jax.ShapeDtypeStruct((HP, HID_CH // 2), jnp.float32),
        ],
    )(x_p, W1, deg_parts)


def _mm2_body(a0_ref, a1_ref, h0_ref, h1_ref, deg_ref, b1_ref, w2_ref,
              g_ref):
    dinv = _dinv_of(deg_ref[...])
    b = b1_ref[...]
    t0 = a0_ref[...] + h0_ref[...]
    t1 = a1_ref[...] + h1_ref[...]
    z0 = jnp.maximum(t0 * dinv[:, None] + b[:, :HID_CH // 2], 0.0)
    z1 = jnp.maximum(t1 * dinv[:, None] + b[:, HID_CH // 2:], 0.0)
    w2 = w2_ref[...]
    h = jnp.dot(z0, w2[:HID_CH // 2], preferred_element_type=jnp.float32)
    h = h + jnp.dot(z1, w2[HID_CH // 2:], preferred_element_type=jnp.float32)
    g_ref[...] = h * dinv[:, None]


def _mm2(a0, a1, h0, h1, deg_parts, b1r, W2):
    grid = (HP // RB,)
    return pl.pallas_call(
        _mm2_body,
        grid=grid,
        in_specs=[
            pl.BlockSpec((RB, HID_CH // 2), lambda i: (i, 0)),
            pl.BlockSpec((RB, HID_CH // 2), lambda i: (i, 0)),
            pl.BlockSpec((RB, HID_CH // 2), lambda i: (i, 0)),
            pl.BlockSpec((RB, HID_CH // 2), lambda i: (i, 0)),
            pl.BlockSpec((NW, RB), lambda i: (0, i)),
            pl.BlockSpec((1, HID_CH), lambda i: (0, 0)),
            pl.BlockSpec((HID_CH, OUT_CH), lambda i: (0, 0)),
        ],
        out_specs=pl.BlockSpec((RB, OUT_CH), lambda i: (i, 0)),
        out_shape=jax.ShapeDtypeStruct((HP, OUT_CH), jnp.float32),
    )(a0, a1, h0, h1, deg_parts, b1r, W2)


def _mm3_body(c0_ref, c1_ref, g_ref2, deg_ref, b2_ref, out_ref):
    dinv = _dinv_of(deg_ref[...])
    o = c0_ref[...] + c1_ref[...] + g_ref2[...]
    out_ref[...] = o * dinv[:, None] + b2_ref[...]


def _mm3(c0, c1, g, deg_parts, b2r):
    grid = (HP // RB,)
    return pl.pallas_call(
        _mm3_body,
        grid=grid,
        in_specs=[
            pl.BlockSpec((RB, OUT_CH), lambda i: (i, 0)),
            pl.BlockSpec((RB, OUT_CH), lambda i: (i, 0)),
            pl.BlockSpec((RB, OUT_CH), lambda i: (i, 0)),
            pl.BlockSpec((NW, RB), lambda i: (0, i)),
            pl.BlockSpec((1, OUT_CH), lambda i: (0, 0)),
        ],
        out_specs=pl.BlockSpec((RB, OUT_CH), lambda i: (i, 0)),
        out_shape=jax.ShapeDtypeStruct((N_NODES, OUT_CH), jnp.float32),
    )(c0, c1, g, deg_parts, b2r)


_deg_kernel = _make_deg()
_agg_hid = _make_agg(HID_CH // 2)
_agg_out = _make_agg_edge_split(OUT_CH)


def kernel(x, edge_index, W1, b1, W2, b2):
    zrows = jnp.zeros((HP // NS, OUT_CH), jnp.float32)

    epairs = _eprep(edge_index.astype(jnp.int32))
    deg_parts = _deg_kernel(epairs)
    h0, h1 = _mm1(x, W1, deg_parts)
    a0, a1 = _agg_hid(h0, h1, zrows, epairs)
    g = _mm2(a0, a1, h0, h1, deg_parts,
             b1.reshape(1, HID_CH), W2)
    c0, c1 = _agg_out(g, zrows, epairs)
    return _mm3(c0, c1, g, deg_parts,
                b2.reshape(1, OUT_CH))


# speedup vs baseline: 1.0016x; 1.0016x over previous
import functools

import jax
import jax.numpy as jnp
from jax import lax
from jax.experimental import pallas as pl
from jax.experimental.pallas import tpu as pltpu
from jax.experimental.pallas import tpu_sc as plsc

N_NODES = 10000
N_EDGES = 160000
IN_CH = 256
HID_CH = 256
OUT_CH = 128

NC = 2
NS = 16
L = 16
NW = NC * NS

HP = 10240
DUMP = N_NODES
EC = 128
EP = 163840

RB = 1024


def _sc_mesh():
    return plsc.VectorSubcoreMesh(core_axis_name="c", subcore_axis_name="s")


_CB = 128


def _eprep_body(ei_ref, ep_ref):
    b = pl.program_id(0)
    e = ei_ref[...].reshape(2, _CB, EC)
    spare = HP - N_NODES
    ch = jax.lax.broadcasted_iota(jnp.int32, (2, _CB, EC), 1)
    lane = jax.lax.broadcasted_iota(jnp.int32, (2, _CB, EC), 2)
    eid = (b * _CB + ch) * EC + lane
    p = eid - N_EDGES
    row = jax.lax.broadcasted_iota(jnp.int32, (2, _CB, EC), 0)
    pad_val = DUMP + ((p + jnp.where(row == 0, spare // 2, 0)) % spare)
    ep_ref[...] = jnp.where(eid >= N_EDGES, pad_val, e)


def _eprep(edge_index):
    nch = EP // EC
    grid = (nch // _CB,)
    return pl.pallas_call(
        _eprep_body,
        grid=grid,
        in_specs=[pl.BlockSpec((2, _CB * EC), lambda i: (0, i))],
        out_specs=pl.BlockSpec((2, _CB, EC), lambda i: (0, i, 0)),
        out_shape=jax.ShapeDtypeStruct((2, nch, EC), jnp.int32),
    )(edge_index)


def _make_deg():
    epw = EP // NW
    nchunks = epw // EC

    @functools.partial(
        pl.kernel,
        out_type=jax.ShapeDtypeStruct((NW, HP), jnp.float32),
        mesh=_sc_mesh(),
        scratch_types=[
            pltpu.VMEM((2 * G, EC), jnp.int32),
            pltpu.VMEM((HP,), jnp.float32),
            pltpu.SemaphoreType.DMA,
        ],
        compiler_params=pltpu.CompilerParams(needs_layout_passes=False),
    )
    def deg_kernel(epairs, deg_hbm, dstb, hist, isem):
        c = lax.axis_index("c")
        s = lax.axis_index("s")
        wid = c * NS + s
        ch0 = wid * nchunks
        ng = nchunks // G
        zeros = jnp.zeros((L,), jnp.float32)

        def zero_body(i, carry):
            for j in range(8):
                hist[pl.ds((i * 8 + j) * L, L)] = zeros
            return carry

        lax.fori_loop(0, HP // L // 8, zero_body, 0)

        ones = jnp.ones((L,), jnp.float32)

        def idx_start(g, gb):
            pltpu.async_copy(epairs.at[1, pl.ds(ch0 + g * G, G)],
                             dstb.at[pl.ds(gb * G, G)], isem)

        def idx_wait(gb):
            pltpu.make_async_copy(epairs.at[1, pl.ds(ch0, G)],
                                  dstb.at[pl.ds(gb * G, G)], isem).wait()

        pltpu.sync_copy(epairs.at[1, pl.ds(ch0, G)],
                        dstb.at[pl.ds(0, G)])
        idx_start(1, 1)

        def group(m, gb, tail=False):
            for j in range(G):
                for l in range(EC // L):
                    idx = dstb[gb * G + j, pl.ds(l * L, L)]
                    plsc.addupdate_scatter(hist, [idx], ones)

            if not tail:
                @pl.when(m < ng - 1)
                def _():
                    idx_wait(gb ^ 1)

                @pl.when(m < ng - 2)
                def _():
                    idx_start(m + 2, gb)

        def body(mm, carry):
            group(2 * mm, 0)
            group(2 * mm + 1, 1)
            return carry

        lax.fori_loop(0, ng // 2, body, 0)
        if ng % 2:
            group(ng - 1, (ng - 1) % 2, tail=True)
        pltpu.sync_copy(hist, deg_hbm.at[wid])

    return deg_kernel


G = 8


def _edge_loop(h, epairs, acc, sidxb, didxb, rows, gsem0, gsem1, isem,
               chunk0, nchunks):
    gsems = (gsem0, gsem1)
    ng = nchunks // G

    def idx_start(g, gb):
        pltpu.async_copy(epairs.at[0, pl.ds(chunk0 + g * G, G)],
                         sidxb.at[gb], isem)
        pltpu.async_copy(epairs.at[1, pl.ds(chunk0 + g * G, G)],
                         didxb.at[gb], isem)

    def idx_wait(gb):
        pltpu.make_async_copy(epairs.at[0, pl.ds(chunk0, G)],
                              sidxb.at[gb], isem).wait()
        pltpu.make_async_copy(epairs.at[1, pl.ds(chunk0, G)],
                              didxb.at[gb], isem).wait()

    def start_gather(b, gb, j):
        pltpu.async_copy(h.at[sidxb.at[gb, j]], rows.at[b], gsems[b])

    def wait_gather(b):
        pltpu.make_async_copy(h.at[sidxb.at[0, 0]], rows.at[b],
                              gsems[b]).wait()

    def scatter(b, gb, j):
        pltpu.sync_copy(rows.at[b], acc.at[didxb.at[gb, j]], add=True)

    pltpu.sync_copy(epairs.at[0, pl.ds(chunk0, G)], sidxb.at[0])
    pltpu.sync_copy(epairs.at[1, pl.ds(chunk0, G)], didxb.at[0])
    idx_start(1, 1)
    start_gather(0, 0, 0)

    def group(m, gb, tail=False):
        for j in range(G):
            rb = j & 1
            wait_gather(rb)
            if j < G - 1:
                start_gather(rb ^ 1, gb, j + 1)
            elif not tail:
                @pl.when(m < ng - 1)
                def _():
                    idx_wait(gb ^ 1)
                    start_gather(rb ^ 1, gb ^ 1, 0)
            scatter(rb, gb, j)

        if not tail:
            @pl.when(m < ng - 2)
            def _():
                idx_start(m + 2, gb)

    def body(mm, carry):
        group(2 * mm, 0)
        group(2 * mm + 1, 1)
        return carry

    lax.fori_loop(0, ng // 2, body, 0)
    if ng % 2:
        group(ng - 1, (ng - 1) % 2, tail=True)


def _agg_scratch(F):
    return [
        pltpu.VMEM((2, G, EC), jnp.int32),
        pltpu.VMEM((2, G, EC), jnp.int32),
        pltpu.VMEM((2, EC, F), jnp.float32),
        pltpu.VMEM_SHARED((HP, F), jnp.float32),
        pltpu.SemaphoreType.DMA,
        pltpu.SemaphoreType.DMA,
        pltpu.SemaphoreType.DMA,
    ]


def _make_agg(F):
    rpw = HP // NS
    nchunks = EP // EC // NS

    @functools.partial(
        pl.kernel,
        out_type=(jax.ShapeDtypeStruct((HP, F), jnp.float32),
                  jax.ShapeDtypeStruct((HP, F), jnp.float32)),
        mesh=_sc_mesh(),
        scratch_types=_agg_scratch(F),
    )
    def agg_kernel(h0, h1, zrows, epairs, o0, o1, sidxb, didxb, rows, acc,
                   gsem0, gsem1, isem):
        c = lax.axis_index("c")
        s = lax.axis_index("s")
        r0 = s * rpw
        chunk0 = s * nchunks

        pltpu.sync_copy(zrows, acc.at[pl.ds(r0, rpw)])
        plsc.subcore_barrier()

        @pl.when(c == 0)
        def _():
            _edge_loop(h0, epairs, acc, sidxb, didxb, rows, gsem0, gsem1,
                       isem, chunk0, nchunks)

        @pl.when(c == 1)
        def _():
            _edge_loop(h1, epairs, acc, sidxb, didxb, rows, gsem0, gsem1,
                       isem, chunk0, nchunks)

        plsc.subcore_barrier()

        def writeback(o):
            pltpu.sync_copy(acc.at[pl.ds(r0, rpw)], o.at[pl.ds(r0, rpw)])

        @pl.when(c == 0)
        def _():
            writeback(o0)

        @pl.when(c == 1)
        def _():
            writeback(o1)

    return agg_kernel


def _make_agg_edge_split(F):
    rpw = HP // NS
    nchunks = EP // EC // NW

    @functools.partial(
        pl.kernel,
        out_type=(jax.ShapeDtypeStruct((HP, F), jnp.float32),
                  jax.ShapeDtypeStruct((HP, F), jnp.float32)),
        mesh=_sc_mesh(),
        scratch_types=_agg_scratch(F),
    )
    def agg_kernel(g, zrows, epairs, o0, o1, sidxb, didxb, rows, acc,
                   gsem0, gsem1, isem):
        c = lax.axis_index("c")
        s = lax.axis_index("s")
        wid = c * NS + s
        r0 = s * rpw

        pltpu.sync_copy(zrows, acc.at[pl.ds(r0, rpw)])
        plsc.subcore_barrier()
        _edge_loop(g, epairs, acc, sidxb, didxb, rows, gsem0, gsem1, isem,
                   wid * nchunks, nchunks)
        plsc.subcore_barrier()

        @pl.when(c == 0)
        def _():
            pltpu.sync_copy(acc.at[pl.ds(r0, rpw)], o0.at[pl.ds(r0, rpw)])

        @pl.when(c == 1)
        def _():
            pltpu.sync_copy(acc.at[pl.ds(r0, rpw)], o1.at[pl.ds(r0, rpw)])

    return agg_kernel


def _dinv_of(deg_blk):
    return lax.rsqrt(1.0 + jnp.sum(deg_blk, axis=0))


def _mm1_body(x_ref, w_ref, deg_ref, h0_ref, h1_ref):
    dinv = _dinv_of(deg_ref[...])
    h = jnp.dot(x_ref[...], w_ref[...], preferred_element_type=jnp.float32)
    h = h * dinv[:, None]
    h0_ref[...] = h[:, :HID_CH // 2]
    h1_ref[...] = h[:, HID_CH // 2:]


def _mm1(x_p, W1, deg_parts):
    grid = (HP // RB,)
    return pl.pallas_call(
        _mm1_body,
        grid=grid,
        in_specs=[
            pl.BlockSpec((RB, IN_CH), lambda i: (i, 0)),
            pl.BlockSpec((IN_CH, HID_CH), lambda i: (0, 0)),
            pl.BlockSpec((NW, RB), lambda i: (0, i)),
        ],
        out_specs=[
            pl.BlockSpec((RB, HID_CH // 2), lambda i: (i, 0)),
            pl.BlockSpec((RB, HID_CH // 2), lambda i: (i, 0)),
        ],
        out_shape=[
            jax.ShapeDtypeStruct((HP, HID_CH // 2), jnp.float32),
            jax.ShapeDtypeStruct((HP, HID_CH // 2), jnp.float32),
        ],
    )(x_p, W1, deg_parts)


def _mm2_body(a0_ref, a1_ref, h0_ref, h1_ref, deg_ref, b1_ref, w2_ref,
              g_ref):
    dinv = _dinv_of(deg_ref[...])
    b = b1_ref[...]
    t0 = a0_ref[...] + h0_ref[...]
    t1 = a1_ref[...] + h1_ref[...]
    z0 = jnp.maximum(t0 * dinv[:, None] + b[:, :HID_CH // 2], 0.0)
    z1 = jnp.maximum(t1 * dinv[:, None] + b[:, HID_CH // 2:], 0.0)
    w2 = w2_ref[...]
    h = jnp.dot(z0, w2[:HID_CH // 2], preferred_element_type=jnp.float32)
    h = h + jnp.dot(z1, w2[HID_CH // 2:], preferred_element_type=jnp.float32)
    g_ref[...] = h * dinv[:, None]


def _mm2(a0, a1, h0, h1, deg_parts, b1r, W2):
    grid = (HP // RB,)
    return pl.pallas_call(
        _mm2_body,
        grid=grid,
        in_specs=[
            pl.BlockSpec((RB, HID_CH // 2), lambda i: (i, 0)),
            pl.BlockSpec((RB, HID_CH // 2), lambda i: (i, 0)),
            pl.BlockSpec((RB, HID_CH // 2), lambda i: (i, 0)),
            pl.BlockSpec((RB, HID_CH // 2), lambda i: (i, 0)),
            pl.BlockSpec((NW, RB), lambda i: (0, i)),
            pl.BlockSpec((1, HID_CH), lambda i: (0, 0)),
            pl.BlockSpec((HID_CH, OUT_CH), lambda i: (0, 0)),
        ],
        out_specs=pl.BlockSpec((RB, OUT_CH), lambda i: (i, 0)),
        out_shape=jax.ShapeDtypeStruct((HP, OUT_CH), jnp.float32),
    )(a0, a1, h0, h1, deg_parts, b1r, W2)


def _mm3_body(c0_ref, c1_ref, g_ref2, deg_ref, b2_ref, out_ref):
    dinv = _dinv_of(deg_ref[...])
    o = c0_ref[...] + c1_ref[...] + g_ref2[...]
    out_ref[...] = o * dinv[:, None] + b2_ref[...]


def _mm3(c0, c1, g, deg_parts, b2r):
    grid = (HP // RB,)
    return pl.pallas_call(
        _mm3_body,
        grid=grid,
        in_specs=[
            pl.BlockSpec((RB, OUT_CH), lambda i: (i, 0)),
            pl.BlockSpec((RB, OUT_CH), lambda i: (i, 0)),
            pl.BlockSpec((RB, OUT_CH), lambda i: (i, 0)),
            pl.BlockSpec((NW, RB), lambda i: (0, i)),
            pl.BlockSpec((1, OUT_CH), lambda i: (0, 0)),
        ],
        out_specs=pl.BlockSpec((RB, OUT_CH), lambda i: (i, 0)),
        out_shape=jax.ShapeDtypeStruct((N_NODES, OUT_CH), jnp.float32),
    )(c0, c1, g, deg_parts, b2r)


_deg_kernel = _make_deg()
_agg_hid = _make_agg(HID_CH // 2)
_agg_out = _make_agg_edge_split(OUT_CH)


def kernel(x, edge_index, W1, b1, W2, b2):
    zrows = jnp.zeros((HP // NS, OUT_CH), jnp.float32)

    epairs = _eprep(edge_index.astype(jnp.int32))
    deg_parts = _deg_kernel(epairs)
    h0, h1 = _mm1(x, W1, deg_parts)
    a0, a1 = _agg_hid(h0, h1, zrows, epairs)
    g = _mm2(a0, a1, h0, h1, deg_parts,
             b1.reshape(1, HID_CH), W2)
    c0, c1 = _agg_out(g, zrows, epairs)
    return _mm3(c0, c1, g, deg_parts,
                b2.reshape(1, OUT_CH))


# speedup vs baseline: 1.0225x; 1.0209x over previous
import functools

import jax
import jax.numpy as jnp
from jax import lax
from jax.experimental import pallas as pl
from jax.experimental.pallas import tpu as pltpu
from jax.experimental.pallas import tpu_sc as plsc

N_NODES = 10000
N_EDGES = 160000
IN_CH = 256
HID_CH = 256
OUT_CH = 128

NC = 2
NS = 16
L = 16
NW = NC * NS

HP = 10240
DUMP = N_NODES
EC = 128
EP = 163840

RB = 2048


def _sc_mesh():
    return plsc.VectorSubcoreMesh(core_axis_name="c", subcore_axis_name="s")


_CB = 128


def _eprep_body(ei_ref, ep_ref):
    b = pl.program_id(0)
    e = ei_ref[...].reshape(2, _CB, EC)
    spare = HP - N_NODES
    ch = jax.lax.broadcasted_iota(jnp.int32, (2, _CB, EC), 1)
    lane = jax.lax.broadcasted_iota(jnp.int32, (2, _CB, EC), 2)
    eid = (b * _CB + ch) * EC + lane
    p = eid - N_EDGES
    row = jax.lax.broadcasted_iota(jnp.int32, (2, _CB, EC), 0)
    pad_val = DUMP + ((p + jnp.where(row == 0, spare // 2, 0)) % spare)
    ep_ref[...] = jnp.where(eid >= N_EDGES, pad_val, e)


def _eprep(edge_index):
    nch = EP // EC
    grid = (nch // _CB,)
    return pl.pallas_call(
        _eprep_body,
        grid=grid,
        in_specs=[pl.BlockSpec((2, _CB * EC), lambda i: (0, i))],
        out_specs=pl.BlockSpec((2, _CB, EC), lambda i: (0, i, 0)),
        out_shape=jax.ShapeDtypeStruct((2, nch, EC), jnp.int32),
    )(edge_index)


def _make_deg():
    epw = EP // NW
    nchunks = epw // EC

    @functools.partial(
        pl.kernel,
        out_type=jax.ShapeDtypeStruct((NW, HP), jnp.float32),
        mesh=_sc_mesh(),
        scratch_types=[
            pltpu.VMEM((2 * G, EC), jnp.int32),
            pltpu.VMEM((HP,), jnp.float32),
            pltpu.SemaphoreType.DMA,
        ],
        compiler_params=pltpu.CompilerParams(needs_layout_passes=False),
    )
    def deg_kernel(epairs, deg_hbm, dstb, hist, isem):
        c = lax.axis_index("c")
        s = lax.axis_index("s")
        wid = c * NS + s
        ch0 = wid * nchunks
        ng = nchunks // G
        zeros = jnp.zeros((L,), jnp.float32)

        def zero_body(i, carry):
            for j in range(8):
                hist[pl.ds((i * 8 + j) * L, L)] = zeros
            return carry

        lax.fori_loop(0, HP // L // 8, zero_body, 0)

        ones = jnp.ones((L,), jnp.float32)

        def idx_start(g, gb):
            pltpu.async_copy(epairs.at[1, pl.ds(ch0 + g * G, G)],
                             dstb.at[pl.ds(gb * G, G)], isem)

        def idx_wait(gb):
            pltpu.make_async_copy(epairs.at[1, pl.ds(ch0, G)],
                                  dstb.at[pl.ds(gb * G, G)], isem).wait()

        pltpu.sync_copy(epairs.at[1, pl.ds(ch0, G)],
                        dstb.at[pl.ds(0, G)])
        idx_start(1, 1)

        def group(m, gb, tail=False):
            for j in range(G):
                for l in range(EC // L):
                    idx = dstb[gb * G + j, pl.ds(l * L, L)]
                    plsc.addupdate_scatter(hist, [idx], ones)

            if not tail:
                @pl.when(m < ng - 1)
                def _():
                    idx_wait(gb ^ 1)

                @pl.when(m < ng - 2)
                def _():
                    idx_start(m + 2, gb)

        def body(mm, carry):
            group(2 * mm, 0)
            group(2 * mm + 1, 1)
            return carry

        lax.fori_loop(0, ng // 2, body, 0)
        if ng % 2:
            group(ng - 1, (ng - 1) % 2, tail=True)
        pltpu.sync_copy(hist, deg_hbm.at[wid])

    return deg_kernel


G = 8


def _edge_loop(h, epairs, acc, sidxb, didxb, rows, gsem0, gsem1, isem,
               chunk0, nchunks):
    gsems = (gsem0, gsem1)
    ng = nchunks // G

    def idx_start(g, gb):
        pltpu.async_copy(epairs.at[0, pl.ds(chunk0 + g * G, G)],
                         sidxb.at[gb], isem)
        pltpu.async_copy(epairs.at[1, pl.ds(chunk0 + g * G, G)],
                         didxb.at[gb], isem)

    def idx_wait(gb):
        pltpu.make_async_copy(epairs.at[0, pl.ds(chunk0, G)],
                              sidxb.at[gb], isem).wait()
        pltpu.make_async_copy(epairs.at[1, pl.ds(chunk0, G)],
                              didxb.at[gb], isem).wait()

    def start_gather(b, gb, j):
        pltpu.async_copy(h.at[sidxb.at[gb, j]], rows.at[b], gsems[b])

    def wait_gather(b):
        pltpu.make_async_copy(h.at[sidxb.at[0, 0]], rows.at[b],
                              gsems[b]).wait()

    def scatter(b, gb, j):
        pltpu.sync_copy(rows.at[b], acc.at[didxb.at[gb, j]], add=True)

    pltpu.sync_copy(epairs.at[0, pl.ds(chunk0, G)], sidxb.at[0])
    pltpu.sync_copy(epairs.at[1, pl.ds(chunk0, G)], didxb.at[0])
    idx_start(1, 1)
    start_gather(0, 0, 0)

    def group(m, gb, tail=False):
        for j in range(G):
            rb = j & 1
            wait_gather(rb)
            if j < G - 1:
                start_gather(rb ^ 1, gb, j + 1)
            elif not tail:
                @pl.when(m < ng - 1)
                def _():
                    idx_wait(gb ^ 1)
                    start_gather(rb ^ 1, gb ^ 1, 0)
            scatter(rb, gb, j)

        if not tail:
            @pl.when(m < ng - 2)
            def _():
                idx_start(m + 2, gb)

    def body(mm, carry):
        group(2 * mm, 0)
        group(2 * mm + 1, 1)
        return carry

    lax.fori_loop(0, ng // 2, body, 0)
    if ng % 2:
        group(ng - 1, (ng - 1) % 2, tail=True)


def _agg_scratch(F):
    return [
        pltpu.VMEM((2, G, EC), jnp.int32),
        pltpu.VMEM((2, G, EC), jnp.int32),
        pltpu.VMEM((2, EC, F), jnp.float32),
        pltpu.VMEM_SHARED((HP, F), jnp.float32),
        pltpu.SemaphoreType.DMA,
        pltpu.SemaphoreType.DMA,
        pltpu.SemaphoreType.DMA,
    ]


def _make_agg(F):
    rpw = HP // NS
    nchunks = EP // EC // NS

    @functools.partial(
        pl.kernel,
        out_type=(jax.ShapeDtypeStruct((HP, F), jnp.float32),
                  jax.ShapeDtypeStruct((HP, F), jnp.float32)),
        mesh=_sc_mesh(),
        scratch_types=_agg_scratch(F),
    )
    def agg_kernel(h0, h1, zrows, epairs, o0, o1, sidxb, didxb, rows, acc,
                   gsem0, gsem1, isem):
        c = lax.axis_index("c")
        s = lax.axis_index("s")
        r0 = s * rpw
        chunk0 = s * nchunks

        pltpu.sync_copy(zrows, acc.at[pl.ds(r0, rpw)])
        plsc.subcore_barrier()

        @pl.when(c == 0)
        def _():
            _edge_loop(h0, epairs, acc, sidxb, didxb, rows, gsem0, gsem1,
                       isem, chunk0, nchunks)

        @pl.when(c == 1)
        def _():
            _edge_loop(h1, epairs, acc, sidxb, didxb, rows, gsem0, gsem1,
                       isem, chunk0, nchunks)

        plsc.subcore_barrier()

        def writeback(o):
            pltpu.sync_copy(acc.at[pl.ds(r0, rpw)], o.at[pl.ds(r0, rpw)])

        @pl.when(c == 0)
        def _():
            writeback(o0)

        @pl.when(c == 1)
        def _():
            writeback(o1)

    return agg_kernel


def _make_agg_edge_split(F):
    rpw = HP // NS
    nchunks = EP // EC // NW

    @functools.partial(
        pl.kernel,
        out_type=(jax.ShapeDtypeStruct((HP, F), jnp.float32),
                  jax.ShapeDtypeStruct((HP, F), jnp.float32)),
        mesh=_sc_mesh(),
        scratch_types=_agg_scratch(F),
    )
    def agg_kernel(g, zrows, epairs, o0, o1, sidxb, didxb, rows, acc,
                   gsem0, gsem1, isem):
        c = lax.axis_index("c")
        s = lax.axis_index("s")
        wid = c * NS + s
        r0 = s * rpw

        pltpu.sync_copy(zrows, acc.at[pl.ds(r0, rpw)])
        plsc.subcore_barrier()
        _edge_loop(g, epairs, acc, sidxb, didxb, rows, gsem0, gsem1, isem,
                   wid * nchunks, nchunks)
        plsc.subcore_barrier()

        @pl.when(c == 0)
        def _():
            pltpu.sync_copy(acc.at[pl.ds(r0, rpw)], o0.at[pl.ds(r0, rpw)])

        @pl.when(c == 1)
        def _():
            pltpu.sync_copy(acc.at[pl.ds(r0, rpw)], o1.at[pl.ds(r0, rpw)])

    return agg_kernel


def _dinv_of(deg_blk):
    return lax.rsqrt(1.0 + jnp.sum(deg_blk, axis=0))


def _mm1_body(x_ref, w_ref, deg_ref, h0_ref, h1_ref):
    dinv = _dinv_of(deg_ref[...])
    h = jnp.dot(x_ref[...], w_ref[...], preferred_element_type=jnp.float32)
    h = h * dinv[:, None]
    h0_ref[...] = h[:, :HID_CH // 2]
    h1_ref[...] = h[:, HID_CH // 2:]


def _mm1(x_p, W1, deg_parts):
    grid = (HP // RB,)
    return pl.pallas_call(
        _mm1_body,
        grid=grid,
        in_specs=[
            pl.BlockSpec((RB, IN_CH), lambda i: (i, 0)),
            pl.BlockSpec((IN_CH, HID_CH), lambda i: (0, 0)),
            pl.BlockSpec((NW, RB), lambda i: (0, i)),
        ],
        out_specs=[
            pl.BlockSpec((RB, HID_CH // 2), lambda i: (i, 0)),
            pl.BlockSpec((RB, HID_CH // 2), lambda i: (i, 0)),
        ],
        out_shape=[
            jax.ShapeDtypeStruct((HP, HID_CH // 2), jnp.float32),
            jax.ShapeDtypeStruct((HP, HID_CH // 2), jnp.float32),
        ],
    )(x_p, W1, deg_parts)


def _mm2_body(a0_ref, a1_ref, h0_ref, h1_ref, deg_ref, b1_ref, w2_ref,
              g_ref):
    dinv = _dinv_of(deg_ref[...])
    b = b1_ref[...]
    t0 = a0_ref[...] + h0_ref[...]
    t1 = a1_ref[...] + h1_ref[...]
    z0 = jnp.maximum(t0 * dinv[:, None] + b[:, :HID_CH // 2], 0.0)
    z1 = jnp.maximum(t1 * dinv[:, None] + b[:, HID_CH // 2:], 0.0)
    w2 = w2_ref[...]
    h = jnp.dot(z0, w2[:HID_CH // 2], preferred_element_type=jnp.float32)
    h = h + jnp.dot(z1, w2[HID_CH // 2:], preferred_element_type=jnp.float32)
    g_ref[...] = h * dinv[:, None]


def _mm2(a0, a1, h0, h1, deg_parts, b1r, W2):
    grid = (HP // RB,)
    return pl.pallas_call(
        _mm2_body,
        grid=grid,
        in_specs=[
            pl.BlockSpec((RB, HID_CH // 2), lambda i: (i, 0)),
            pl.BlockSpec((RB, HID_CH // 2), lambda i: (i, 0)),
            pl.BlockSpec((RB, HID_CH // 2), lambda i: (i, 0)),
            pl.BlockSpec((RB, HID_CH // 2), lambda i: (i, 0)),
            pl.BlockSpec((NW, RB), lambda i: (0, i)),
            pl.BlockSpec((1, HID_CH), lambda i: (0, 0)),
            pl.BlockSpec((HID_CH, OUT_CH), lambda i: (0, 0)),
        ],
        out_specs=pl.BlockSpec((RB, OUT_CH), lambda i: (i, 0)),
        out_shape=jax.ShapeDtypeStruct((HP, OUT_CH), jnp.float32),
    )(a0, a1, h0, h1, deg_parts, b1r, W2)


def _mm3_body(c0_ref, c1_ref, g_ref2, deg_ref, b2_ref, out_ref):
    dinv = _dinv_of(deg_ref[...])
    o = c0_ref[...] + c1_ref[...] + g_ref2[...]
    out_ref[...] = o * dinv[:, None] + b2_ref[...]


def _mm3(c0, c1, g, deg_parts, b2r):
    grid = (HP // RB,)
    return pl.pallas_call(
        _mm3_body,
        grid=grid,
        in_specs=[
            pl.BlockSpec((RB, OUT_CH), lambda i: (i, 0)),
            pl.BlockSpec((RB, OUT_CH), lambda i: (i, 0)),
            pl.BlockSpec((RB, OUT_CH), lambda i: (i, 0)),
            pl.BlockSpec((NW, RB), lambda i: (0, i)),
            pl.BlockSpec((1, OUT_CH), lambda i: (0, 0)),
        ],
        out_specs=pl.BlockSpec((RB, OUT_CH), lambda i: (i, 0)),
        out_shape=jax.ShapeDtypeStruct((N_NODES, OUT_CH), jnp.float32),
    )(c0, c1, g, deg_parts, b2r)


_deg_kernel = _make_deg()
_agg_hid = _make_agg(HID_CH // 2)
_agg_out = _make_agg_edge_split(OUT_CH)


def kernel(x, edge_index, W1, b1, W2, b2):
    zrows = jnp.zeros((HP // NS, OUT_CH), jnp.float32)

    epairs = _eprep(edge_index.astype(jnp.int32))
    deg_parts = _deg_kernel(epairs)
    h0, h1 = _mm1(x, W1, deg_parts)
    a0, a1 = _agg_hid(h0, h1, zrows, epairs)
    g = _mm2(a0, a1, h0, h1, deg_parts,
             b1.reshape(1, HID_CH), W2)
    c0, c1 = _agg_out(g, zrows, epairs)
    return _mm3(c0, c1, g, deg_parts,
                b2.reshape(1, OUT_CH))
